# Initial kernel scaffold; baseline (speedup 1.0000x reference)
import jax
import jax.numpy as jnp
from jax.experimental import pallas as pl


def kernel(masks):
    raise NotImplementedError("write your pallas kernel here")


# speedup vs baseline: 1.0084x; 1.0084x over previous
import functools

import jax
import jax.numpy as jnp
from jax import lax
from jax.experimental import pallas as pl
from jax.experimental.pallas import tpu as pltpu
from jax.experimental.pallas import tpu_sc as plsc

B, S, LANES = 4, 8192, 2048
NW = 32
LC = LANES * B // NW
NCHUNK = LC // 16
ST = 128
NTILES = S // ST

_mesh = plsc.VectorSubcoreMesh(core_axis_name="c", subcore_axis_name="s")


@functools.partial(
    pl.kernel,
    out_type=jax.ShapeDtypeStruct((B, S, LANES), jnp.float32),
    mesh=_mesh,
    scratch_types=[
        pltpu.VMEM((ST, LC), jnp.float32),
    ],
)
def _cumsum_sc(x_hbm, out_hbm, buf):
    wid = lax.axis_index("s") * 2 + lax.axis_index("c")
    b = wid // (NW // B)
    l0 = (wid % (NW // B)) * LC

    def tile_body(t, carries):
        s0 = t * ST
        pltpu.sync_copy(x_hbm.at[b, pl.ds(s0, ST), pl.ds(l0, LC)], buf)

        def row_body(r, cs):
            new = []
            for j in range(NCHUNK):
                c = cs[j] + buf[r, pl.ds(j * 16, 16)]
                buf[r, pl.ds(j * 16, 16)] = c
                new.append(c)
            return tuple(new)

        carries = lax.fori_loop(0, ST, row_body, carries, unroll=4)
        pltpu.sync_copy(buf, out_hbm.at[b, pl.ds(s0, ST), pl.ds(l0, LC)])
        return carries

    zeros = tuple(jnp.zeros((16,), jnp.float32) for _ in range(NCHUNK))
    lax.fori_loop(0, NTILES, tile_body, zeros)


def kernel(masks):
    return _cumsum_sc(masks)


# speedup vs baseline: 1.5295x; 1.5167x over previous
import functools

import jax
import jax.numpy as jnp
from jax import lax
from jax.experimental import pallas as pl
from jax.experimental.pallas import tpu as pltpu
from jax.experimental.pallas import tpu_sc as plsc

B, S, LANES = 4, 8192, 2048
NW = 32
LC = LANES * B // NW
NCHUNK = LC // 16
ST = 64
NTILES = S // ST
NPAIRS = NTILES // 2

_mesh = plsc.VectorSubcoreMesh(core_axis_name="c", subcore_axis_name="s")


@functools.partial(
    pl.kernel,
    out_type=jax.ShapeDtypeStruct((B, S, LANES), jnp.float32),
    mesh=_mesh,
    scratch_types=[
        pltpu.VMEM((2, ST, LC), jnp.float32),
        pltpu.VMEM((2, ST, LC), jnp.float32),
        pltpu.SemaphoreType.DMA,
        pltpu.SemaphoreType.DMA,
        pltpu.SemaphoreType.DMA,
        pltpu.SemaphoreType.DMA,
    ],
)
def _cumsum_sc(x_hbm, out_hbm, inbuf, outbuf, isem0, isem1, osem0, osem1):
    wid = lax.axis_index("s") * 2 + lax.axis_index("c")
    b = wid // (NW // B)
    l0 = (wid % (NW // B)) * LC
    insems = (isem0, isem1)
    outsems = (osem0, osem1)

    def in_copy(t, phase):
        src = x_hbm.at[b, pl.ds(t * ST, ST), pl.ds(l0, LC)]
        return pltpu.make_async_copy(src, inbuf.at[phase], insems[phase])

    def out_copy(t, phase):
        dst = out_hbm.at[b, pl.ds(t * ST, ST), pl.ds(l0, LC)]
        return pltpu.make_async_copy(outbuf.at[phase], dst, outsems[phase])

    for phase in range(2):
        in_copy(phase, phase).start()

    def pair_body(tp, carries):
        for phase in range(2):
            t = 2 * tp + phase
            in_copy(t, phase).wait()

            @pl.when(tp >= 1)
            def _wait_prev_out(phase=phase, t=t):
                out_copy(t - 2, phase).wait()

            def row_body(r, cs, phase=phase):
                new = []
                for j in range(NCHUNK):
                    c = cs[j] + inbuf[phase, r, pl.ds(j * 16, 16)]
                    outbuf[phase, r, pl.ds(j * 16, 16)] = c
                    new.append(c)
                return tuple(new)

            carries = lax.fori_loop(0, ST, row_body, carries, unroll=4)
            out_copy(t, phase).start()

            @pl.when(tp + 1 < NPAIRS)
            def _prefetch(phase=phase, t=t):
                in_copy(t + 2, phase).start()

        return carries

    lax.fori_loop(0, NPAIRS, pair_body,
                  tuple(jnp.zeros((16,), jnp.float32) for _ in range(NCHUNK)))

    for phase in range(2):
        out_copy(NTILES - 2 + phase, phase).wait()


def kernel(masks):
    return _cumsum_sc(masks)
